# baseline (speedup 1.0000x reference)
import functools

import jax
import jax.numpy as jnp
from jax import lax
from jax.experimental import pallas as pl
from jax.experimental.pallas import tpu as pltpu
from jax.experimental.pallas import tpu_sc as plsc

K = 10
NUM_STEPS = 2
KP = K + 1
PAD = 16

SC_CORES = 2
SC_TILES = 16
NW = SC_CORES * SC_TILES


def _edge_agg_body(r_hbm, src_hbm, dst_hbm, zeros_hbm, out_hbm,
                   idx_s_v, idx_d_v, rows_v, acc_sh, sem):
    c = lax.axis_index("c")
    s = lax.axis_index("s")
    w = c * SC_TILES + s
    n = acc_sh.shape[0]
    rpt = (n // SC_TILES) // 8 * 8
    last = n - rpt * (SC_TILES - 1)

    @pl.when(s < SC_TILES - 1)
    def _():
        pltpu.sync_copy(zeros_hbm.at[pl.ds(0, rpt)],
                        acc_sh.at[pl.ds(s * rpt, rpt)])

    @pl.when(s == SC_TILES - 1)
    def _():
        pltpu.sync_copy(zeros_hbm, acc_sh.at[pl.ds((SC_TILES - 1) * rpt, last)])

    pltpu.sync_copy(src_hbm.at[w], idx_s_v)
    pltpu.sync_copy(dst_hbm.at[w], idx_d_v)
    plsc.subcore_barrier()
    nc = idx_s_v.shape[0]

    def body(j, carry):
        pltpu.async_copy(r_hbm.at[idx_s_v.at[j]], rows_v, sem).wait()
        pltpu.sync_copy(rows_v, acc_sh.at[idx_d_v.at[j]], add=True)
        return carry

    lax.fori_loop(0, nc, body, 0, unroll=False)
    plsc.subcore_barrier()

    @pl.when(s < SC_TILES - 1)
    def _():
        pltpu.sync_copy(acc_sh.at[pl.ds(s * rpt, rpt)],
                        out_hbm.at[c, pl.ds(s * rpt, rpt)])

    @pl.when(s == SC_TILES - 1)
    def _():
        pltpu.sync_copy(acc_sh.at[pl.ds((SC_TILES - 1) * rpt, last)],
                        out_hbm.at[c, pl.ds((SC_TILES - 1) * rpt, last)])


def _edge_agg(r, edges, chunk=125):
    n, d = r.shape
    e = edges.shape[1]
    nc = e // (NW * chunk)
    assert nc * NW * chunk == e, (e, chunk)
    src = edges[0].reshape(NW, nc, chunk)
    dst = edges[1].reshape(NW, nc, chunk)
    rpt = (n // SC_TILES) // 8 * 8
    zeros = jnp.zeros((n - rpt * (SC_TILES - 1), d), jnp.float32)
    mesh = plsc.VectorSubcoreMesh(core_axis_name="c", subcore_axis_name="s")
    f = pl.kernel(
        _edge_agg_body,
        out_type=jax.ShapeDtypeStruct((SC_CORES, n, d), jnp.float32),
        mesh=mesh,
        scratch_types=[
            pltpu.VMEM((nc, chunk), jnp.int32),
            pltpu.VMEM((nc, chunk), jnp.int32),
            pltpu.VMEM((chunk, d), jnp.float32),
            pltpu.VMEM_SHARED((n, d), jnp.float32),
            pltpu.SemaphoreType.DMA,
        ],
    )
    return f(r, src, dst, zeros)


def _topk_body(s_ref, shk_ref, idx_ref, p_ref, *, n1):
    vals = s_ref[...]
    r = vals.shape[0]
    col = jax.lax.broadcasted_iota(jnp.int32, vals.shape, 1)
    neg = jnp.float32(-1e30)
    vcols = []
    icols = []
    for _ in range(KP):
        m = jnp.max(vals, axis=1)
        eq = vals == m[:, None]
        idx = jnp.min(jnp.where(eq, col, n1), axis=1)
        vcols.append(m[:, None])
        icols.append(idx[:, None])
        vals = jnp.where(col == idx[:, None], neg, vals)
    vcols.append(jnp.full((r, PAD - KP), neg, jnp.float32))
    icols.append(jnp.zeros((r, PAD - KP), jnp.int32))
    shk = jnp.concatenate(vcols, axis=1) * jnp.float32(NUM_STEPS)
    shk_ref[...] = shk
    idx_ref[...] = jnp.concatenate(icols, axis=1)
    mx = jnp.max(shk, axis=1, keepdims=True)
    e = jnp.exp(shk - mx)
    p_ref[...] = e / jnp.sum(e, axis=1, keepdims=True)


def _topk(s_hat, block_rows):
    n0, n1 = s_hat.shape
    grid = n0 // block_rows
    return pl.pallas_call(
        functools.partial(_topk_body, n1=n1),
        grid=(grid,),
        in_specs=[pl.BlockSpec((block_rows, n1), lambda i: (i, 0))],
        out_specs=[
            pl.BlockSpec((block_rows, PAD), lambda i: (i, 0)),
            pl.BlockSpec((block_rows, PAD), lambda i: (i, 0)),
            pl.BlockSpec((block_rows, PAD), lambda i: (i, 0)),
        ],
        out_shape=[
            jax.ShapeDtypeStruct((n0, PAD), jnp.float32),
            jax.ShapeDtypeStruct((n0, PAD), jnp.int32),
            jax.ShapeDtypeStruct((n0, PAD), jnp.float32),
        ],
    )(s_hat)


_GDNUMS = lax.GatherDimensionNumbers(
    offset_dims=(), collapsed_slice_dims=(0,), start_index_map=(0,))


def _lane_shuffle(vec, idx):
    return lax.gather(vec, idx.reshape(16, 1), _GDNUMS, (1,),
                      mode=lax.GatherScatterMode.PROMISE_IN_BOUNDS)


def _lane_bcast(vec, lane):
    return _lane_shuffle(vec, jnp.full((16,), lane, jnp.int32))


def _all_sum(v):
    lanes = lax.iota(jnp.int32, 16)
    for sh in (1, 2, 4, 8):
        v = v + _lane_shuffle(v, lanes ^ sh)
    return v


def _all_max(v):
    lanes = lax.iota(jnp.int32, 16)
    for sh in (1, 2, 4, 8):
        v = jnp.maximum(v, _lane_shuffle(v, lanes ^ sh))
    return v


def _wscatter_body(rs_hbm, rowf_hbm, colf_hbm, valf_hbm, zeros_hbm, out_hbm,
                   rowv, colv, valv, rows_v, acc_sh, sem):
    c = lax.axis_index("c")
    s = lax.axis_index("s")
    w = c * SC_TILES + s
    n = acc_sh.shape[0]
    rpt = (n // SC_TILES) // 8 * 8
    last = n - rpt * (SC_TILES - 1)

    @pl.when(s < SC_TILES - 1)
    def _():
        pltpu.sync_copy(zeros_hbm.at[pl.ds(0, rpt)],
                        acc_sh.at[pl.ds(s * rpt, rpt)])

    @pl.when(s == SC_TILES - 1)
    def _():
        pltpu.sync_copy(zeros_hbm, acc_sh.at[pl.ds((SC_TILES - 1) * rpt, last)])

    pltpu.sync_copy(rowf_hbm.at[w], rowv)
    pltpu.sync_copy(colf_hbm.at[w], colv)
    pltpu.sync_copy(valf_hbm.at[w], valv)
    plsc.subcore_barrier()
    nc, chunk = rowv.shape

    def body(j, carry):
        pltpu.async_copy(rs_hbm.at[rowv.at[j]], rows_v, sem).wait()
        for g in range(chunk // 16):
            vv = valv[j, pl.ds(g * 16, 16)]
            for l in range(16):
                b = _lane_bcast(vv, l)
                r = g * 16 + l
                for q in range(8):
                    rows_v[r, pl.ds(q * 16, 16)] = (
                        rows_v[r, pl.ds(q * 16, 16)] * b)
        pltpu.sync_copy(rows_v, acc_sh.at[colv.at[j]], add=True)
        return carry

    lax.fori_loop(0, nc, body, 0, unroll=False)
    plsc.subcore_barrier()

    @pl.when(s < SC_TILES - 1)
    def _():
        pltpu.sync_copy(acc_sh.at[pl.ds(s * rpt, rpt)],
                        out_hbm.at[c, pl.ds(s * rpt, rpt)])

    @pl.when(s == SC_TILES - 1)
    def _():
        pltpu.sync_copy(acc_sh.at[pl.ds((SC_TILES - 1) * rpt, last)],
                        out_hbm.at[c, pl.ds((SC_TILES - 1) * rpt, last)])


def _wscatter(rs, rowf, colf, valf, n_out):
    n, d = rs.shape
    m = rowf.shape[0]
    chunk = 80
    nc = m // (NW * chunk)
    assert nc * NW * chunk == m, (m, chunk)
    rowf = rowf.reshape(NW, nc, chunk)
    colf = colf.reshape(NW, nc, chunk)
    valf = valf.reshape(NW, nc, chunk)
    rpt = (n_out // SC_TILES) // 8 * 8
    zeros = jnp.zeros((n_out - rpt * (SC_TILES - 1), d), jnp.float32)
    mesh = plsc.VectorSubcoreMesh(core_axis_name="c", subcore_axis_name="s")
    f = pl.kernel(
        _wscatter_body,
        out_type=jax.ShapeDtypeStruct((SC_CORES, n_out, d), jnp.float32),
        mesh=mesh,
        scratch_types=[
            pltpu.VMEM((nc, chunk), jnp.int32),
            pltpu.VMEM((nc, chunk), jnp.int32),
            pltpu.VMEM((nc, chunk), jnp.float32),
            pltpu.VMEM((chunk, d), jnp.float32),
            pltpu.VMEM_SHARED((n_out, d), jnp.float32),
            pltpu.SemaphoreType.DMA,
        ],
    )
    return f(rs, rowf, colf, valf, zeros)


def _sim_softmax_body(os_hbm, ot_hbm, idxf_hbm, shkf_hbm, shk_out, p_out,
                      idx_v, shk_v, os_v, gat_v, shko_v, po_v, sem):
    c = lax.axis_index("c")
    s = lax.axis_index("s")
    w = c * SC_TILES + s
    nchunks = idxf_hbm.shape[0] // 128
    per = (nchunks + NW - 1) // NW
    lane = jax.lax.iota(jnp.int32, 16)

    def body(jj, carry):
        chunkid = w + NW * jj

        @pl.when(chunkid < nchunks)
        def _():
            pltpu.sync_copy(idxf_hbm.at[pl.ds(chunkid * 128, 128)], idx_v)
            pltpu.sync_copy(shkf_hbm.at[pl.ds(chunkid * 128, 128)], shk_v)
            pltpu.sync_copy(os_hbm.at[pl.ds(chunkid * 8, 8)], os_v)
            pltpu.async_copy(ot_hbm.at[idx_v], gat_v, sem).wait()
            for r in range(8):
                simrow = jnp.zeros((16,), jnp.float32)
                for k in range(KP):
                    a = jnp.zeros((16,), jnp.float32)
                    for q in range(8):
                        a = a + (os_v[r, pl.ds(q * 16, 16)] *
                                 gat_v[r * 16 + k, pl.ds(q * 16, 16)])
                    dk = _all_sum(a)
                    simrow = jnp.where(lane == k, simrow + dk, simrow)
                shk_row = shk_v[pl.ds(r * 16, 16)] + simrow
                m = _all_max(shk_row)
                e = jnp.exp(shk_row - m)
                ssum = _all_sum(e)
                p_row = e / ssum
                shko_v[pl.ds(r * 16, 16)] = shk_row
                po_v[pl.ds(r * 16, 16)] = p_row
            pltpu.sync_copy(shko_v, shk_out.at[pl.ds(chunkid * 128, 128)])
            pltpu.sync_copy(po_v, p_out.at[pl.ds(chunkid * 128, 128)])

        return carry

    lax.fori_loop(0, per, body, 0, unroll=False)


def _sim_softmax(o_s, o_t, idx_pad, shk_pad):
    n, d = o_s.shape
    idxf = idx_pad.reshape(-1)
    shkf = shk_pad.reshape(-1)
    mesh = plsc.VectorSubcoreMesh(core_axis_name="c", subcore_axis_name="s")
    f = pl.kernel(
        _sim_softmax_body,
        out_type=(jax.ShapeDtypeStruct((n * PAD,), jnp.float32),
                  jax.ShapeDtypeStruct((n * PAD,), jnp.float32)),
        mesh=mesh,
        scratch_types=[
            pltpu.VMEM((128,), jnp.int32),
            pltpu.VMEM((128,), jnp.float32),
            pltpu.VMEM((8, d), jnp.float32),
            pltpu.VMEM((128, d), jnp.float32),
            pltpu.VMEM((128,), jnp.float32),
            pltpu.VMEM((128,), jnp.float32),
            pltpu.SemaphoreType.DMA,
        ],
    )
    shkf_new, pf = f(o_s, o_t, idxf, shkf)
    return shkf_new.reshape(n, PAD), pf.reshape(n, PAD)


def _psi2_mm_body(r_ref, a0_ref, a1_ref, ws_ref, wn_ref, o_ref):
    agg = a0_ref[0] + a1_ref[0]
    o = jnp.dot(r_ref[...], ws_ref[...], preferred_element_type=jnp.float32)
    o = o + jnp.dot(agg, wn_ref[...], preferred_element_type=jnp.float32)
    o = jnp.maximum(o, 0.0)
    nrm = jnp.sqrt(jnp.sum(o * o, axis=1, keepdims=True))
    o_ref[...] = o / jnp.maximum(nrm, 1e-12)


def _psi2_mm(r, aggpair, w_self, w_nbr):
    n, d = r.shape
    b = 1000
    return pl.pallas_call(
        _psi2_mm_body,
        grid=(n // b,),
        in_specs=[
            pl.BlockSpec((b, d), lambda i: (i, 0)),
            pl.BlockSpec((1, b, d), lambda i: (0, i, 0)),
            pl.BlockSpec((1, b, d), lambda i: (1, i, 0)),
            pl.BlockSpec((d, d), lambda i: (0, 0)),
            pl.BlockSpec((d, d), lambda i: (0, 0)),
        ],
        out_specs=pl.BlockSpec((b, d), lambda i: (i, 0)),
        out_shape=jax.ShapeDtypeStruct((n, d), jnp.float32),
    )(r, aggpair, aggpair, w_self, w_nbr)


def _sum2_body(x_ref, o_ref):
    o_ref[...] = x_ref[0] + x_ref[1]


def _sum2(pair):
    _, n, d = pair.shape
    b = 1000
    return pl.pallas_call(
        _sum2_body,
        grid=(n // b,),
        in_specs=[pl.BlockSpec((2, b, d), lambda i: (0, i, 0))],
        out_specs=pl.BlockSpec((b, d), lambda i: (i, 0)),
        out_shape=jax.ShapeDtypeStruct((n, d), jnp.float32),
    )(pair)


def _psi2(r, edges, w_self, w_nbr):
    ap = _edge_agg(r, edges)
    return _psi2_mm(r, ap, w_self, w_nbr)


def _l2norm(x):
    return x / jnp.clip(jnp.linalg.norm(x, axis=-1, keepdims=True), 1e-12, None)


def kernel(S_hat, edges_s, edges_t, W_self, W_nbr):
    n0, n1 = S_hat.shape
    rnd_dim = W_self.shape[0]
    block_rows = 200 if n0 % 200 == 0 else 8

    shk_pad, idx_pad, p_pad = _topk(S_hat, block_rows)

    m = n0 * KP
    cpw = 80 * NW
    mp = -(-m // cpw) * cpw
    zpad_i = jnp.zeros((mp - m,), jnp.int32)
    zpad_f = jnp.zeros((mp - m,), jnp.float32)
    rowf = jnp.concatenate(
        [jnp.repeat(jnp.arange(n0, dtype=jnp.int32), KP), zpad_i])
    colf = jnp.concatenate([idx_pad[:, :KP].reshape(-1), zpad_i])

    rkey = jax.random.key(42)
    for step in range(NUM_STEPS):
        r_s = jax.random.normal(jax.random.fold_in(rkey, step), (n0, rnd_dim),
                                dtype=S_hat.dtype)
        valf = jnp.concatenate([p_pad[:, :KP].reshape(-1), zpad_f])
        rtp = _wscatter(r_s, rowf, colf, valf, n1)
        r_t = _sum2(rtp)
        o_s = _psi2(r_s, edges_s, W_self, W_nbr)
        o_t = _psi2(r_t, edges_t, W_self, W_nbr)
        shk_pad, p_pad = _sim_softmax(o_s, o_t, idx_pad, shk_pad)
    return p_pad[:, :KP]


# speedup vs baseline: 1.9307x; 1.9307x over previous
import functools

import jax
import jax.numpy as jnp
from jax import lax
from jax.experimental import pallas as pl
from jax.experimental.pallas import tpu as pltpu
from jax.experimental.pallas import tpu_sc as plsc

K = 10
NUM_STEPS = 2
KP = K + 1
PAD = 16

SC_CORES = 2
SC_TILES = 16
NW = SC_CORES * SC_TILES


def _edge_agg_body(r_hbm, src_hbm, dst_hbm, zeros_hbm, out_hbm,
                   idx_s_v, idx_d_v, rows_v, acc_sh, sem):
    c = lax.axis_index("c")
    s = lax.axis_index("s")
    w = c * SC_TILES + s
    n = acc_sh.shape[0]
    rpt = (n // SC_TILES) // 8 * 8
    last = n - rpt * (SC_TILES - 1)

    @pl.when(s < SC_TILES - 1)
    def _():
        pltpu.sync_copy(zeros_hbm.at[pl.ds(0, rpt)],
                        acc_sh.at[pl.ds(s * rpt, rpt)])

    @pl.when(s == SC_TILES - 1)
    def _():
        pltpu.sync_copy(zeros_hbm, acc_sh.at[pl.ds((SC_TILES - 1) * rpt, last)])

    pltpu.sync_copy(src_hbm.at[w], idx_s_v)
    pltpu.sync_copy(dst_hbm.at[w], idx_d_v)
    plsc.subcore_barrier()
    nc = idx_s_v.shape[0]

    def body(j, carry):
        pltpu.async_copy(r_hbm.at[idx_s_v.at[j]], rows_v, sem).wait()
        pltpu.sync_copy(rows_v, acc_sh.at[idx_d_v.at[j]], add=True)
        return carry

    lax.fori_loop(0, nc, body, 0, unroll=False)
    plsc.subcore_barrier()

    @pl.when(s < SC_TILES - 1)
    def _():
        pltpu.sync_copy(acc_sh.at[pl.ds(s * rpt, rpt)],
                        out_hbm.at[c, pl.ds(s * rpt, rpt)])

    @pl.when(s == SC_TILES - 1)
    def _():
        pltpu.sync_copy(acc_sh.at[pl.ds((SC_TILES - 1) * rpt, last)],
                        out_hbm.at[c, pl.ds((SC_TILES - 1) * rpt, last)])


def _edge_agg(r, edges, chunk=125):
    n, d = r.shape
    e = edges.shape[1]
    nc = e // (NW * chunk)
    assert nc * NW * chunk == e, (e, chunk)
    src = edges[0].reshape(NW, nc, chunk)
    dst = edges[1].reshape(NW, nc, chunk)
    rpt = (n // SC_TILES) // 8 * 8
    zeros = jnp.zeros((n - rpt * (SC_TILES - 1), d), jnp.float32)
    mesh = plsc.VectorSubcoreMesh(core_axis_name="c", subcore_axis_name="s")
    f = pl.kernel(
        _edge_agg_body,
        out_type=jax.ShapeDtypeStruct((SC_CORES, n, d), jnp.float32),
        mesh=mesh,
        scratch_types=[
            pltpu.VMEM((nc, chunk), jnp.int32),
            pltpu.VMEM((nc, chunk), jnp.int32),
            pltpu.VMEM((chunk, d), jnp.float32),
            pltpu.VMEM_SHARED((n, d), jnp.float32),
            pltpu.SemaphoreType.DMA,
        ],
    )
    return f(r, src, dst, zeros)


def _topk_body(s_ref, shk_ref, idx_ref, p_ref, *, n1):
    vals = s_ref[...]
    r = vals.shape[0]
    col = jax.lax.broadcasted_iota(jnp.int32, vals.shape, 1)
    neg = jnp.float32(-1e30)
    vcols = []
    icols = []
    for _ in range(KP):
        m = jnp.max(vals, axis=1)
        eq = vals == m[:, None]
        idx = jnp.min(jnp.where(eq, col, n1), axis=1)
        vcols.append(m[:, None])
        icols.append(idx[:, None])
        vals = jnp.where(col == idx[:, None], neg, vals)
    vcols.append(jnp.full((r, PAD - KP), neg, jnp.float32))
    icols.append(jnp.zeros((r, PAD - KP), jnp.int32))
    shk = jnp.concatenate(vcols, axis=1) * jnp.float32(NUM_STEPS)
    shk_ref[...] = shk
    idx_ref[...] = jnp.concatenate(icols, axis=1)
    mx = jnp.max(shk, axis=1, keepdims=True)
    e = jnp.exp(shk - mx)
    p_ref[...] = e / jnp.sum(e, axis=1, keepdims=True)


def _topk(s_hat, block_rows):
    n0, n1 = s_hat.shape
    grid = n0 // block_rows
    return pl.pallas_call(
        functools.partial(_topk_body, n1=n1),
        grid=(grid,),
        in_specs=[pl.BlockSpec((block_rows, n1), lambda i: (i, 0))],
        out_specs=[
            pl.BlockSpec((block_rows, PAD), lambda i: (i, 0)),
            pl.BlockSpec((block_rows, PAD), lambda i: (i, 0)),
            pl.BlockSpec((block_rows, PAD), lambda i: (i, 0)),
        ],
        out_shape=[
            jax.ShapeDtypeStruct((n0, PAD), jnp.float32),
            jax.ShapeDtypeStruct((n0, PAD), jnp.int32),
            jax.ShapeDtypeStruct((n0, PAD), jnp.float32),
        ],
    )(s_hat)


_GDNUMS = lax.GatherDimensionNumbers(
    offset_dims=(), collapsed_slice_dims=(0,), start_index_map=(0,))


def _lane_shuffle(vec, idx):
    return lax.gather(vec, idx.reshape(16, 1), _GDNUMS, (1,),
                      mode=lax.GatherScatterMode.PROMISE_IN_BOUNDS)


def _lane_bcast(vec, lane):
    return _lane_shuffle(vec, jnp.full((16,), lane, jnp.int32))


def _all_sum(v):
    lanes = lax.iota(jnp.int32, 16)
    for sh in (1, 2, 4, 8):
        v = v + _lane_shuffle(v, lanes ^ sh)
    return v


def _all_max(v):
    lanes = lax.iota(jnp.int32, 16)
    for sh in (1, 2, 4, 8):
        v = jnp.maximum(v, _lane_shuffle(v, lanes ^ sh))
    return v


def _wscatter_body(rs_hbm, rowf_hbm, colf_hbm, valf_hbm, zeros_hbm, out_hbm,
                   rowv, colv, valv, rows_v, acc_sh, sem):
    c = lax.axis_index("c")
    s = lax.axis_index("s")
    w = c * SC_TILES + s
    n = acc_sh.shape[0]
    rpt = (n // SC_TILES) // 8 * 8
    last = n - rpt * (SC_TILES - 1)

    @pl.when(s < SC_TILES - 1)
    def _():
        pltpu.sync_copy(zeros_hbm.at[pl.ds(0, rpt)],
                        acc_sh.at[pl.ds(s * rpt, rpt)])

    @pl.when(s == SC_TILES - 1)
    def _():
        pltpu.sync_copy(zeros_hbm, acc_sh.at[pl.ds((SC_TILES - 1) * rpt, last)])

    pltpu.sync_copy(rowf_hbm.at[w], rowv)
    pltpu.sync_copy(colf_hbm.at[w], colv)
    pltpu.sync_copy(valf_hbm.at[w], valv)
    plsc.subcore_barrier()
    nc, chunk = rowv.shape

    def body(j, carry):
        pltpu.async_copy(rs_hbm.at[rowv.at[j]], rows_v, sem).wait()
        for g in range(chunk // 16):
            vv = valv[j, pl.ds(g * 16, 16)]
            for l in range(16):
                b = _lane_bcast(vv, l)
                r = g * 16 + l
                for q in range(8):
                    rows_v[r, pl.ds(q * 16, 16)] = (
                        rows_v[r, pl.ds(q * 16, 16)] * b)
        pltpu.sync_copy(rows_v, acc_sh.at[colv.at[j]], add=True)
        return carry

    lax.fori_loop(0, nc, body, 0, unroll=False)
    plsc.subcore_barrier()

    @pl.when(s < SC_TILES - 1)
    def _():
        pltpu.sync_copy(acc_sh.at[pl.ds(s * rpt, rpt)],
                        out_hbm.at[c, pl.ds(s * rpt, rpt)])

    @pl.when(s == SC_TILES - 1)
    def _():
        pltpu.sync_copy(acc_sh.at[pl.ds((SC_TILES - 1) * rpt, last)],
                        out_hbm.at[c, pl.ds((SC_TILES - 1) * rpt, last)])


def _wscatter(rs, rowf, colf, valf, n_out):
    n, d = rs.shape
    m = rowf.shape[0]
    chunk = 80
    nc = m // (NW * chunk)
    assert nc * NW * chunk == m, (m, chunk)
    rowf = rowf.reshape(NW, nc, chunk)
    colf = colf.reshape(NW, nc, chunk)
    valf = valf.reshape(NW, nc, chunk)
    rpt = (n_out // SC_TILES) // 8 * 8
    zeros = jnp.zeros((n_out - rpt * (SC_TILES - 1), d), jnp.float32)
    mesh = plsc.VectorSubcoreMesh(core_axis_name="c", subcore_axis_name="s")
    f = pl.kernel(
        _wscatter_body,
        out_type=jax.ShapeDtypeStruct((SC_CORES, n_out, d), jnp.float32),
        mesh=mesh,
        scratch_types=[
            pltpu.VMEM((nc, chunk), jnp.int32),
            pltpu.VMEM((nc, chunk), jnp.int32),
            pltpu.VMEM((nc, chunk), jnp.float32),
            pltpu.VMEM((chunk, d), jnp.float32),
            pltpu.VMEM_SHARED((n_out, d), jnp.float32),
            pltpu.SemaphoreType.DMA,
        ],
    )
    return f(rs, rowf, colf, valf, zeros)


def _gather_body(tab_hbm, colv_hbm, out_hbm, colv, rows_v, sem):
    c = lax.axis_index("c")
    s = lax.axis_index("s")
    w = c * SC_TILES + s
    nc, chunk = colv.shape
    pltpu.sync_copy(colv_hbm.at[w], colv)

    def body(j, carry):
        pltpu.async_copy(tab_hbm.at[colv.at[j]], rows_v, sem).wait()
        pltpu.sync_copy(rows_v, out_hbm.at[pl.ds((w * nc + j) * chunk, chunk)])
        return carry

    lax.fori_loop(0, nc, body, 0, unroll=False)


def _gather_rows(tab, colf):
    n, d = tab.shape
    mp = colf.shape[0]
    chunk = 80
    nc = mp // (NW * chunk)
    assert nc * NW * chunk == mp, (mp, chunk)
    colv = colf.reshape(NW, nc, chunk)
    mesh = plsc.VectorSubcoreMesh(core_axis_name="c", subcore_axis_name="s")
    f = pl.kernel(
        _gather_body,
        out_type=jax.ShapeDtypeStruct((mp, d), jnp.float32),
        mesh=mesh,
        scratch_types=[
            pltpu.VMEM((nc, chunk), jnp.int32),
            pltpu.VMEM((chunk, d), jnp.float32),
            pltpu.SemaphoreType.DMA,
        ],
    )
    return f(tab, colv)


def _sim_softmax_tc_body(gat_ref, os_ref, shk_ref, shko_ref, p_ref, acc_ref):
    k = pl.program_id(1)
    b = os_ref.shape[0]
    col = jnp.sum(gat_ref[...] * os_ref[...], axis=1, keepdims=True)
    lanes = lax.broadcasted_iota(jnp.int32, (b, PAD), 1)

    @pl.when(k == 0)
    def _():
        acc_ref[...] = jnp.zeros((b, PAD), jnp.float32)

    acc_ref[...] = jnp.where(lanes == k, acc_ref[...] + col, acc_ref[...])

    @pl.when(k == KP - 1)
    def _():
        shk_new = shk_ref[...] + acc_ref[...]
        mx = jnp.max(shk_new, axis=1, keepdims=True)
        e = jnp.exp(shk_new - mx)
        shko_ref[...] = shk_new
        p_ref[...] = e / jnp.sum(e, axis=1, keepdims=True)


def _sim_softmax(gat_flat, o_s, shk_pad):
    n, d = o_s.shape
    b = 400
    nb = n // b
    return pl.pallas_call(
        _sim_softmax_tc_body,
        grid=(nb, KP),
        in_specs=[
            pl.BlockSpec((b, d), lambda i, k: (k * nb + i, 0)),
            pl.BlockSpec((b, d), lambda i, k: (i, 0)),
            pl.BlockSpec((b, PAD), lambda i, k: (i, 0)),
        ],
        out_specs=[
            pl.BlockSpec((b, PAD), lambda i, k: (i, 0)),
            pl.BlockSpec((b, PAD), lambda i, k: (i, 0)),
        ],
        out_shape=[
            jax.ShapeDtypeStruct((n, PAD), jnp.float32),
            jax.ShapeDtypeStruct((n, PAD), jnp.float32),
        ],
        scratch_shapes=[pltpu.VMEM((b, PAD), jnp.float32)],
    )(gat_flat, o_s, shk_pad)


def _psi2_mm_body(r_ref, a0_ref, a1_ref, ws_ref, wn_ref, o_ref):
    agg = a0_ref[0] + a1_ref[0]
    o = jnp.dot(r_ref[...], ws_ref[...], preferred_element_type=jnp.float32)
    o = o + jnp.dot(agg, wn_ref[...], preferred_element_type=jnp.float32)
    o = jnp.maximum(o, 0.0)
    nrm = jnp.sqrt(jnp.sum(o * o, axis=1, keepdims=True))
    o_ref[...] = o / jnp.maximum(nrm, 1e-12)


def _psi2_mm(r, aggpair, w_self, w_nbr):
    n, d = r.shape
    b = 1000
    return pl.pallas_call(
        _psi2_mm_body,
        grid=(n // b,),
        in_specs=[
            pl.BlockSpec((b, d), lambda i: (i, 0)),
            pl.BlockSpec((1, b, d), lambda i: (0, i, 0)),
            pl.BlockSpec((1, b, d), lambda i: (1, i, 0)),
            pl.BlockSpec((d, d), lambda i: (0, 0)),
            pl.BlockSpec((d, d), lambda i: (0, 0)),
        ],
        out_specs=pl.BlockSpec((b, d), lambda i: (i, 0)),
        out_shape=jax.ShapeDtypeStruct((n, d), jnp.float32),
    )(r, aggpair, aggpair, w_self, w_nbr)


def _sum2_body(x_ref, o_ref):
    o_ref[...] = x_ref[0] + x_ref[1]


def _sum2(pair):
    _, n, d = pair.shape
    b = 1000
    return pl.pallas_call(
        _sum2_body,
        grid=(n // b,),
        in_specs=[pl.BlockSpec((2, b, d), lambda i: (0, i, 0))],
        out_specs=pl.BlockSpec((b, d), lambda i: (i, 0)),
        out_shape=jax.ShapeDtypeStruct((n, d), jnp.float32),
    )(pair)


def _psi2(r, edges, w_self, w_nbr):
    ap = _edge_agg(r, edges)
    return _psi2_mm(r, ap, w_self, w_nbr)


def _l2norm(x):
    return x / jnp.clip(jnp.linalg.norm(x, axis=-1, keepdims=True), 1e-12, None)


def kernel(S_hat, edges_s, edges_t, W_self, W_nbr):
    n0, n1 = S_hat.shape
    rnd_dim = W_self.shape[0]
    block_rows = 200 if n0 % 200 == 0 else 8

    shk_pad, idx_pad, p_pad = _topk(S_hat, block_rows)

    m = n0 * KP
    cpw = 80 * NW
    mp = -(-m // cpw) * cpw
    zpad_i = jnp.zeros((mp - m,), jnp.int32)
    zpad_f = jnp.zeros((mp - m,), jnp.float32)
    rowf = jnp.concatenate(
        [jnp.repeat(jnp.arange(n0, dtype=jnp.int32), KP), zpad_i])
    colf = jnp.concatenate([idx_pad[:, :KP].reshape(-1), zpad_i])
    import math
    g = cpw * 400 // math.gcd(cpw, 400)
    mp2 = -(-m // g) * g
    colf_k = jnp.concatenate(
        [idx_pad[:, :KP].T.reshape(-1),
         jnp.zeros((mp2 - m,), jnp.int32)])

    rkey = jax.random.key(42)
    for step in range(NUM_STEPS):
        r_s = jax.random.normal(jax.random.fold_in(rkey, step), (n0, rnd_dim),
                                dtype=S_hat.dtype)
        valf = jnp.concatenate([p_pad[:, :KP].reshape(-1), zpad_f])
        rtp = _wscatter(r_s, rowf, colf, valf, n1)
        r_t = _sum2(rtp)
        o_s = _psi2(r_s, edges_s, W_self, W_nbr)
        o_t = _psi2(r_t, edges_t, W_self, W_nbr)
        gat = _gather_rows(o_t, colf_k)
        shk_pad, p_pad = _sim_softmax(gat, o_s, shk_pad)
    return p_pad[:, :KP]


# speedup vs baseline: 1.9648x; 1.0177x over previous
import functools

import jax
import jax.numpy as jnp
from jax import lax
from jax.experimental import pallas as pl
from jax.experimental.pallas import tpu as pltpu
from jax.experimental.pallas import tpu_sc as plsc

K = 10
NUM_STEPS = 2
KP = K + 1
PAD = 16

SC_CORES = 2
SC_TILES = 16
NW = SC_CORES * SC_TILES


def _edge_agg_body(r_hbm, src_hbm, dst_hbm, zeros_hbm, out_hbm,
                   idx_s_v, idx_d_v, rows_v, acc_sh, sem):
    c = lax.axis_index("c")
    s = lax.axis_index("s")
    w = c * SC_TILES + s
    n = acc_sh.shape[0]
    rpt = (n // SC_TILES) // 8 * 8
    last = n - rpt * (SC_TILES - 1)

    @pl.when(s < SC_TILES - 1)
    def _():
        pltpu.sync_copy(zeros_hbm.at[pl.ds(0, rpt)],
                        acc_sh.at[pl.ds(s * rpt, rpt)])

    @pl.when(s == SC_TILES - 1)
    def _():
        pltpu.sync_copy(zeros_hbm, acc_sh.at[pl.ds((SC_TILES - 1) * rpt, last)])

    pltpu.sync_copy(src_hbm.at[w], idx_s_v)
    pltpu.sync_copy(dst_hbm.at[w], idx_d_v)
    plsc.subcore_barrier()
    nc = idx_s_v.shape[0]

    def body(j, carry):
        pltpu.async_copy(r_hbm.at[idx_s_v.at[j]], rows_v, sem).wait()
        pltpu.sync_copy(rows_v, acc_sh.at[idx_d_v.at[j]], add=True)
        return carry

    lax.fori_loop(0, nc, body, 0, unroll=False)
    plsc.subcore_barrier()

    @pl.when(s < SC_TILES - 1)
    def _():
        pltpu.sync_copy(acc_sh.at[pl.ds(s * rpt, rpt)],
                        out_hbm.at[c, pl.ds(s * rpt, rpt)])

    @pl.when(s == SC_TILES - 1)
    def _():
        pltpu.sync_copy(acc_sh.at[pl.ds((SC_TILES - 1) * rpt, last)],
                        out_hbm.at[c, pl.ds((SC_TILES - 1) * rpt, last)])


def _edge_agg(r, edges, chunk=125):
    n, d = r.shape
    e = edges.shape[1]
    nc = e // (NW * chunk)
    assert nc * NW * chunk == e, (e, chunk)
    src = edges[0].reshape(NW, nc, chunk)
    dst = edges[1].reshape(NW, nc, chunk)
    rpt = (n // SC_TILES) // 8 * 8
    zeros = jnp.zeros((n - rpt * (SC_TILES - 1), d), jnp.float32)
    mesh = plsc.VectorSubcoreMesh(core_axis_name="c", subcore_axis_name="s")
    f = pl.kernel(
        _edge_agg_body,
        out_type=jax.ShapeDtypeStruct((SC_CORES, n, d), jnp.float32),
        mesh=mesh,
        scratch_types=[
            pltpu.VMEM((nc, chunk), jnp.int32),
            pltpu.VMEM((nc, chunk), jnp.int32),
            pltpu.VMEM((chunk, d), jnp.float32),
            pltpu.VMEM_SHARED((n, d), jnp.float32),
            pltpu.SemaphoreType.DMA,
        ],
    )
    return f(r, src, dst, zeros)


def _topk_body(s_ref, shk_ref, idx_ref, p_ref, *, n1):
    vals = s_ref[...]
    r = vals.shape[0]
    col = jax.lax.broadcasted_iota(jnp.int32, vals.shape, 1)
    neg = jnp.float32(-1e30)
    vcols = []
    icols = []
    for _ in range(KP):
        m = jnp.max(vals, axis=1)
        eq = vals == m[:, None]
        idx = jnp.min(jnp.where(eq, col, n1), axis=1)
        vcols.append(m[:, None])
        icols.append(idx[:, None])
        vals = jnp.where(eq, neg, vals)
    vcols.append(jnp.full((r, PAD - KP), neg, jnp.float32))
    icols.append(jnp.zeros((r, PAD - KP), jnp.int32))
    shk = jnp.concatenate(vcols, axis=1) * jnp.float32(NUM_STEPS)
    shk_ref[...] = shk
    idx_ref[...] = jnp.concatenate(icols, axis=1)
    mx = jnp.max(shk, axis=1, keepdims=True)
    e = jnp.exp(shk - mx)
    p_ref[...] = e / jnp.sum(e, axis=1, keepdims=True)


def _topk(s_hat, block_rows):
    n0, n1 = s_hat.shape
    grid = n0 // block_rows
    return pl.pallas_call(
        functools.partial(_topk_body, n1=n1),
        grid=(grid,),
        in_specs=[pl.BlockSpec((block_rows, n1), lambda i: (i, 0))],
        out_specs=[
            pl.BlockSpec((block_rows, PAD), lambda i: (i, 0)),
            pl.BlockSpec((block_rows, PAD), lambda i: (i, 0)),
            pl.BlockSpec((block_rows, PAD), lambda i: (i, 0)),
        ],
        out_shape=[
            jax.ShapeDtypeStruct((n0, PAD), jnp.float32),
            jax.ShapeDtypeStruct((n0, PAD), jnp.int32),
            jax.ShapeDtypeStruct((n0, PAD), jnp.float32),
        ],
    )(s_hat)


_GDNUMS = lax.GatherDimensionNumbers(
    offset_dims=(), collapsed_slice_dims=(0,), start_index_map=(0,))


def _lane_shuffle(vec, idx):
    return lax.gather(vec, idx.reshape(16, 1), _GDNUMS, (1,),
                      mode=lax.GatherScatterMode.PROMISE_IN_BOUNDS)


def _lane_bcast(vec, lane):
    return _lane_shuffle(vec, jnp.full((16,), lane, jnp.int32))


def _all_sum(v):
    lanes = lax.iota(jnp.int32, 16)
    for sh in (1, 2, 4, 8):
        v = v + _lane_shuffle(v, lanes ^ sh)
    return v


def _all_max(v):
    lanes = lax.iota(jnp.int32, 16)
    for sh in (1, 2, 4, 8):
        v = jnp.maximum(v, _lane_shuffle(v, lanes ^ sh))
    return v


def _wscatter_body(rs_hbm, rowf_hbm, colf_hbm, valf_hbm, zeros_hbm, out_hbm,
                   rowv, colv, valv, rows_v, acc_sh, sem):
    c = lax.axis_index("c")
    s = lax.axis_index("s")
    w = c * SC_TILES + s
    n = acc_sh.shape[0]
    rpt = (n // SC_TILES) // 8 * 8
    last = n - rpt * (SC_TILES - 1)

    @pl.when(s < SC_TILES - 1)
    def _():
        pltpu.sync_copy(zeros_hbm.at[pl.ds(0, rpt)],
                        acc_sh.at[pl.ds(s * rpt, rpt)])

    @pl.when(s == SC_TILES - 1)
    def _():
        pltpu.sync_copy(zeros_hbm, acc_sh.at[pl.ds((SC_TILES - 1) * rpt, last)])

    pltpu.sync_copy(rowf_hbm.at[w], rowv)
    pltpu.sync_copy(colf_hbm.at[w], colv)
    pltpu.sync_copy(valf_hbm.at[w], valv)
    plsc.subcore_barrier()
    nc, chunk = rowv.shape

    def body(j, carry):
        pltpu.async_copy(rs_hbm.at[rowv.at[j]], rows_v, sem).wait()
        for g in range(chunk // 16):
            vv = valv[j, pl.ds(g * 16, 16)]
            for l in range(16):
                b = _lane_bcast(vv, l)
                r = g * 16 + l
                for q in range(8):
                    rows_v[r, pl.ds(q * 16, 16)] = (
                        rows_v[r, pl.ds(q * 16, 16)] * b)
        pltpu.sync_copy(rows_v, acc_sh.at[colv.at[j]], add=True)
        return carry

    lax.fori_loop(0, nc, body, 0, unroll=False)
    plsc.subcore_barrier()

    @pl.when(s < SC_TILES - 1)
    def _():
        pltpu.sync_copy(acc_sh.at[pl.ds(s * rpt, rpt)],
                        out_hbm.at[c, pl.ds(s * rpt, rpt)])

    @pl.when(s == SC_TILES - 1)
    def _():
        pltpu.sync_copy(acc_sh.at[pl.ds((SC_TILES - 1) * rpt, last)],
                        out_hbm.at[c, pl.ds((SC_TILES - 1) * rpt, last)])


def _wscatter(rs, rowf, colf, valf, n_out):
    n, d = rs.shape
    m = rowf.shape[0]
    chunk = 80
    nc = m // (NW * chunk)
    assert nc * NW * chunk == m, (m, chunk)
    rowf = rowf.reshape(NW, nc, chunk)
    colf = colf.reshape(NW, nc, chunk)
    valf = valf.reshape(NW, nc, chunk)
    rpt = (n_out // SC_TILES) // 8 * 8
    zeros = jnp.zeros((n_out - rpt * (SC_TILES - 1), d), jnp.float32)
    mesh = plsc.VectorSubcoreMesh(core_axis_name="c", subcore_axis_name="s")
    f = pl.kernel(
        _wscatter_body,
        out_type=jax.ShapeDtypeStruct((SC_CORES, n_out, d), jnp.float32),
        mesh=mesh,
        scratch_types=[
            pltpu.VMEM((nc, chunk), jnp.int32),
            pltpu.VMEM((nc, chunk), jnp.int32),
            pltpu.VMEM((nc, chunk), jnp.float32),
            pltpu.VMEM((chunk, d), jnp.float32),
            pltpu.VMEM_SHARED((n_out, d), jnp.float32),
            pltpu.SemaphoreType.DMA,
        ],
    )
    return f(rs, rowf, colf, valf, zeros)


def _gather_body(tab_hbm, colv_hbm, out_hbm, colv, rows_v, sem):
    c = lax.axis_index("c")
    s = lax.axis_index("s")
    w = c * SC_TILES + s
    nc, chunk = colv.shape
    pltpu.sync_copy(colv_hbm.at[w], colv)

    def body(j, carry):
        pltpu.async_copy(tab_hbm.at[colv.at[j]], rows_v, sem).wait()
        pltpu.sync_copy(rows_v, out_hbm.at[pl.ds((w * nc + j) * chunk, chunk)])
        return carry

    lax.fori_loop(0, nc, body, 0, unroll=False)


def _gather_rows(tab, colf):
    n, d = tab.shape
    mp = colf.shape[0]
    chunk = 80
    nc = mp // (NW * chunk)
    assert nc * NW * chunk == mp, (mp, chunk)
    colv = colf.reshape(NW, nc, chunk)
    mesh = plsc.VectorSubcoreMesh(core_axis_name="c", subcore_axis_name="s")
    f = pl.kernel(
        _gather_body,
        out_type=jax.ShapeDtypeStruct((mp, d), jnp.float32),
        mesh=mesh,
        scratch_types=[
            pltpu.VMEM((nc, chunk), jnp.int32),
            pltpu.VMEM((chunk, d), jnp.float32),
            pltpu.SemaphoreType.DMA,
        ],
    )
    return f(tab, colv)


def _sim_softmax_tc_body(gat_ref, os_ref, shk_ref, shko_ref, p_ref, acc_ref):
    k = pl.program_id(1)
    b = os_ref.shape[0]
    col = jnp.sum(gat_ref[...] * os_ref[...], axis=1, keepdims=True)
    lanes = lax.broadcasted_iota(jnp.int32, (b, PAD), 1)

    @pl.when(k == 0)
    def _():
        acc_ref[...] = jnp.zeros((b, PAD), jnp.float32)

    acc_ref[...] = jnp.where(lanes == k, acc_ref[...] + col, acc_ref[...])

    @pl.when(k == KP - 1)
    def _():
        shk_new = shk_ref[...] + acc_ref[...]
        mx = jnp.max(shk_new, axis=1, keepdims=True)
        e = jnp.exp(shk_new - mx)
        shko_ref[...] = shk_new
        p_ref[...] = e / jnp.sum(e, axis=1, keepdims=True)


def _sim_softmax(gat_flat, o_s, shk_pad):
    n, d = o_s.shape
    b = 400
    nb = n // b
    return pl.pallas_call(
        _sim_softmax_tc_body,
        grid=(nb, KP),
        in_specs=[
            pl.BlockSpec((b, d), lambda i, k: (k * nb + i, 0)),
            pl.BlockSpec((b, d), lambda i, k: (i, 0)),
            pl.BlockSpec((b, PAD), lambda i, k: (i, 0)),
        ],
        out_specs=[
            pl.BlockSpec((b, PAD), lambda i, k: (i, 0)),
            pl.BlockSpec((b, PAD), lambda i, k: (i, 0)),
        ],
        out_shape=[
            jax.ShapeDtypeStruct((n, PAD), jnp.float32),
            jax.ShapeDtypeStruct((n, PAD), jnp.float32),
        ],
        scratch_shapes=[pltpu.VMEM((b, PAD), jnp.float32)],
    )(gat_flat, o_s, shk_pad)


def _psi2_mm_body(r_ref, a0_ref, a1_ref, ws_ref, wn_ref, o_ref):
    agg = a0_ref[0] + a1_ref[0]
    o = jnp.dot(r_ref[...], ws_ref[...], preferred_element_type=jnp.float32)
    o = o + jnp.dot(agg, wn_ref[...], preferred_element_type=jnp.float32)
    o = jnp.maximum(o, 0.0)
    nrm = jnp.sqrt(jnp.sum(o * o, axis=1, keepdims=True))
    o_ref[...] = o / jnp.maximum(nrm, 1e-12)


def _psi2_mm(r, aggpair, w_self, w_nbr):
    n, d = r.shape
    b = 1000
    return pl.pallas_call(
        _psi2_mm_body,
        grid=(n // b,),
        in_specs=[
            pl.BlockSpec((b, d), lambda i: (i, 0)),
            pl.BlockSpec((1, b, d), lambda i: (0, i, 0)),
            pl.BlockSpec((1, b, d), lambda i: (1, i, 0)),
            pl.BlockSpec((d, d), lambda i: (0, 0)),
            pl.BlockSpec((d, d), lambda i: (0, 0)),
        ],
        out_specs=pl.BlockSpec((b, d), lambda i: (i, 0)),
        out_shape=jax.ShapeDtypeStruct((n, d), jnp.float32),
    )(r, aggpair, aggpair, w_self, w_nbr)


def _sum2_body(x_ref, o_ref):
    o_ref[...] = x_ref[0] + x_ref[1]


def _sum2(pair):
    _, n, d = pair.shape
    b = 1000
    return pl.pallas_call(
        _sum2_body,
        grid=(n // b,),
        in_specs=[pl.BlockSpec((2, b, d), lambda i: (0, i, 0))],
        out_specs=pl.BlockSpec((b, d), lambda i: (i, 0)),
        out_shape=jax.ShapeDtypeStruct((n, d), jnp.float32),
    )(pair)


def _psi2(r, edges, w_self, w_nbr):
    ap = _edge_agg(r, edges)
    return _psi2_mm(r, ap, w_self, w_nbr)


def _l2norm(x):
    return x / jnp.clip(jnp.linalg.norm(x, axis=-1, keepdims=True), 1e-12, None)


def kernel(S_hat, edges_s, edges_t, W_self, W_nbr):
    n0, n1 = S_hat.shape
    rnd_dim = W_self.shape[0]
    block_rows = 80 if n0 % 80 == 0 else 8

    shk_pad, idx_pad, p_pad = _topk(S_hat, block_rows)

    m = n0 * KP
    cpw = 80 * NW
    mp = -(-m // cpw) * cpw
    zpad_i = jnp.zeros((mp - m,), jnp.int32)
    zpad_f = jnp.zeros((mp - m,), jnp.float32)
    rowf = jnp.concatenate(
        [jnp.repeat(jnp.arange(n0, dtype=jnp.int32), KP), zpad_i])
    colf = jnp.concatenate([idx_pad[:, :KP].reshape(-1), zpad_i])
    import math
    g = cpw * 400 // math.gcd(cpw, 400)
    mp2 = -(-m // g) * g
    colf_k = jnp.concatenate(
        [idx_pad[:, :KP].T.reshape(-1),
         jnp.zeros((mp2 - m,), jnp.int32)])

    rkey = jax.random.key(42)
    for step in range(NUM_STEPS):
        r_s = jax.random.normal(jax.random.fold_in(rkey, step), (n0, rnd_dim),
                                dtype=S_hat.dtype)
        valf = jnp.concatenate([p_pad[:, :KP].reshape(-1), zpad_f])
        rtp = _wscatter(r_s, rowf, colf, valf, n1)
        r_t = _sum2(rtp)
        o_s = _psi2(r_s, edges_s, W_self, W_nbr)
        o_t = _psi2(r_t, edges_t, W_self, W_nbr)
        gat = _gather_rows(o_t, colf_k)
        shk_pad, p_pad = _sim_softmax(gat, o_s, shk_pad)
    return p_pad[:, :KP]
